# baseline (device time: 46477 ns/iter reference)
import jax
import jax.numpy as jnp
from jax import lax
from jax.experimental import pallas as pl
from jax.experimental.pallas import tpu as pltpu

N_DEV = 8
BLK = 512

_WAIT_ORDER = [1, 3, 4, 2, 5, 7, 6]


def kernel(x):
    m, n = x.shape

    def body(
        x_ref, out_ref, xv, sbuf, rbuf, vout,
        in_sems, out_sems, send_sems, recv_sems,
    ):
        me = lax.axis_index("i")

        own = pltpu.make_async_copy(
            x_ref.at[:, pl.ds(me * BLK, BLK)],
            out_ref.at[pl.ds(me * BLK, BLK), :],
            out_sems.at[N_DEV - 1],
        )
        own.start()

        loads = {}
        for d in _WAIT_ORDER:
            col = lax.bitwise_xor(me, d)
            ld = pltpu.make_async_copy(
                x_ref.at[:, pl.ds(col * BLK, BLK)],
                xv.at[:, pl.ds(col * BLK, BLK)],
                in_sems.at[d - 1],
            )
            ld.start()
            loads[d] = ld

        rdmas = {}
        for d in _WAIT_ORDER:
            col = lax.bitwise_xor(me, d)
            loads[d].wait()
            sbuf[:, pl.ds(col * BLK, BLK)] = (
                xv[:, pl.ds(col * BLK, BLK)].astype(jnp.bfloat16)
            )
            rdma = pltpu.make_async_remote_copy(
                src_ref=sbuf.at[:, pl.ds(col * BLK, BLK)],
                dst_ref=rbuf.at[d],
                send_sem=send_sems.at[d - 1],
                recv_sem=recv_sems.at[d - 1],
                device_id=(col,),
                device_id_type=pl.DeviceIdType.MESH,
            )
            rdma.start()
            rdmas[d] = rdma

        stores = {}
        for d in _WAIT_ORDER:
            rdmas[d].wait_recv()
            src = lax.bitwise_xor(me, d)
            vout[d - 1, :, :] = rbuf[d].astype(jnp.float32)
            st = pltpu.make_async_copy(
                vout.at[d - 1],
                out_ref.at[pl.ds(src * BLK, BLK), :],
                out_sems.at[d - 1],
            )
            st.start()
            stores[d] = st

        for d in _WAIT_ORDER:
            rdmas[d].wait_send()
            stores[d].wait()
        own.wait()

    return pl.pallas_call(
        body,
        out_shape=jax.ShapeDtypeStruct((N_DEV * m, n // N_DEV), x.dtype),
        in_specs=[pl.BlockSpec(memory_space=pl.ANY)],
        out_specs=pl.BlockSpec(memory_space=pl.ANY),
        scratch_shapes=[
            pltpu.VMEM((m, n), jnp.float32),
            pltpu.VMEM((m, n), jnp.bfloat16),
            pltpu.VMEM((N_DEV, BLK, BLK), jnp.bfloat16),
            pltpu.VMEM((N_DEV - 1, BLK, BLK), jnp.float32),
            pltpu.SemaphoreType.DMA((N_DEV - 1,)),
            pltpu.SemaphoreType.DMA((N_DEV,)),
            pltpu.SemaphoreType.DMA((N_DEV - 1,)),
            pltpu.SemaphoreType.DMA((N_DEV - 1,)),
        ],
    )(x)


# device time: 42524 ns/iter; 1.0930x vs baseline; 1.0930x over previous
import jax
import jax.numpy as jnp
from jax import lax
from jax.experimental import pallas as pl
from jax.experimental.pallas import tpu as pltpu

N_DEV = 8
BLK = 512

_ISSUE_ORDER = [6, 2, 5, 7, 1, 3, 4]
_WAIT_ORDER = [1, 3, 4, 2, 5, 7, 6]


def kernel(x):
    m, n = x.shape

    def body(x_ref, out_ref, sbuf, rbuf, vout, out_sems, send_sems, recv_sems):
        me = lax.axis_index("i")

        rdmas = {}
        for d in _ISSUE_ORDER:
            col = lax.bitwise_xor(me, d)
            sbuf[:, pl.ds(col * BLK, BLK)] = (
                x_ref[:, pl.ds(col * BLK, BLK)].astype(jnp.bfloat16)
            )
            rdma = pltpu.make_async_remote_copy(
                src_ref=sbuf.at[:, pl.ds(col * BLK, BLK)],
                dst_ref=rbuf.at[d],
                send_sem=send_sems.at[d - 1],
                recv_sem=recv_sems.at[d - 1],
                device_id=(col,),
                device_id_type=pl.DeviceIdType.MESH,
            )
            rdma.start()
            rdmas[d] = rdma

        own = pltpu.make_async_copy(
            x_ref.at[:, pl.ds(me * BLK, BLK)],
            out_ref.at[pl.ds(me * BLK, BLK), :],
            out_sems.at[N_DEV - 1],
        )
        own.start()

        stores = {}
        for d in _WAIT_ORDER:
            rdmas[d].wait_recv()
            src = lax.bitwise_xor(me, d)
            vout[d - 1, :, :] = rbuf[d].astype(jnp.float32)
            st = pltpu.make_async_copy(
                vout.at[d - 1],
                out_ref.at[pl.ds(src * BLK, BLK), :],
                out_sems.at[d - 1],
            )
            st.start()
            stores[d] = st

        for d in _WAIT_ORDER:
            rdmas[d].wait_send()
            stores[d].wait()
        own.wait()

    return pl.pallas_call(
        body,
        out_shape=jax.ShapeDtypeStruct((N_DEV * m, n // N_DEV), x.dtype),
        in_specs=[pl.BlockSpec(memory_space=pltpu.VMEM)],
        out_specs=pl.BlockSpec(memory_space=pl.ANY),
        scratch_shapes=[
            pltpu.VMEM((m, n), jnp.bfloat16),
            pltpu.VMEM((N_DEV, BLK, BLK), jnp.bfloat16),
            pltpu.VMEM((N_DEV - 1, BLK, BLK), jnp.float32),
            pltpu.SemaphoreType.DMA((N_DEV,)),
            pltpu.SemaphoreType.DMA((N_DEV - 1,)),
            pltpu.SemaphoreType.DMA((N_DEV - 1,)),
        ],
    )(x)


# device time: 38772 ns/iter; 1.1987x vs baseline; 1.0968x over previous
import jax
import jax.numpy as jnp
from jax import lax
from jax.experimental import pallas as pl
from jax.experimental.pallas import tpu as pltpu

N_DEV = 8
BLK = 512

_OWN_SENDS = [
    (3, 6, 6),
    (4, 1, 8),
    (5, 3, 9),
    (6, 4, 10),
    (0, 1, 1),
    (1, 3, 3),
    (2, 4, 4),
]
_FORWARDS = [
    (7, 8, 4, 3, 2),
    (8, 9, 5, 4, 7),
    (9, 10, 6, 1, 5),
]
_FINALS = [(1, 0), (3, 1), (4, 2), (6, 3), (2, 7), (7, 8), (5, 9)]

_N_RDMA = 10


def kernel(x):
    m, n = x.shape

    def body(x_ref, out_ref, sbuf, rbuf, vout, out_sems, send_sems, recv_sems):
        me = lax.axis_index("i")

        def remote_copy(src_ref, dst_ref, e, dev_mask):
            return pltpu.make_async_remote_copy(
                src_ref=src_ref,
                dst_ref=dst_ref,
                send_sem=send_sems.at[e],
                recv_sem=recv_sems.at[e],
                device_id=(lax.bitwise_xor(me, dev_mask),),
                device_id_type=pl.DeviceIdType.MESH,
            )

        rdmas = {}
        for e, dev_mask, slot in _OWN_SENDS:
            blk_mask = 2 if e == 4 else 7 if e == 5 else 5 if e == 6 else dev_mask
            col = lax.bitwise_xor(me, blk_mask)
            sbuf[:, pl.ds(col * BLK, BLK)] = (
                x_ref[:, pl.ds(col * BLK, BLK)].astype(jnp.bfloat16)
            )
            rdma = remote_copy(
                sbuf.at[:, pl.ds(col * BLK, BLK)], rbuf.at[slot], e, dev_mask
            )
            rdma.start()
            rdmas[e] = rdma

        own = pltpu.make_async_copy(
            x_ref.at[:, pl.ds(me * BLK, BLK)],
            out_ref.at[pl.ds(me * BLK, BLK), :],
            out_sems.at[N_DEV - 1],
        )
        own.start()

        for e, t_slot, r_enum, dev_mask, f_slot in _FORWARDS:
            remote_copy(rbuf.at[t_slot], rbuf.at[t_slot], r_enum, dev_mask).wait_recv()
            fwd = remote_copy(rbuf.at[t_slot], rbuf.at[f_slot], e, dev_mask)
            fwd.start()
            rdmas[e] = fwd

        stores = []
        for mm, r_enum in _FINALS:
            remote_copy(rbuf.at[mm], rbuf.at[mm], r_enum, 1).wait_recv()
            src = lax.bitwise_xor(me, mm)
            vout[mm - 1, :, :] = rbuf[mm].astype(jnp.float32)
            st = pltpu.make_async_copy(
                vout.at[mm - 1],
                out_ref.at[pl.ds(src * BLK, BLK), :],
                out_sems.at[mm - 1],
            )
            st.start()
            stores.append(st)

        for e in range(_N_RDMA):
            rdmas[e].wait_send()
        for st in stores:
            st.wait()
        own.wait()

    return pl.pallas_call(
        body,
        out_shape=jax.ShapeDtypeStruct((N_DEV * m, n // N_DEV), x.dtype),
        in_specs=[pl.BlockSpec(memory_space=pltpu.VMEM)],
        out_specs=pl.BlockSpec(memory_space=pl.ANY),
        scratch_shapes=[
            pltpu.VMEM((m, n), jnp.bfloat16),
            pltpu.VMEM((11, BLK, BLK), jnp.bfloat16),
            pltpu.VMEM((N_DEV - 1, BLK, BLK), jnp.float32),
            pltpu.SemaphoreType.DMA((N_DEV,)),
            pltpu.SemaphoreType.DMA((_N_RDMA,)),
            pltpu.SemaphoreType.DMA((_N_RDMA,)),
        ],
    )(x)
